# baseline (speedup 1.0000x reference)
import functools

import jax
import jax.numpy as jnp
from jax import lax
from jax.experimental import pallas as pl
from jax.experimental.pallas import tpu as pltpu
from jax.experimental.pallas import tpu_sc as plsc

VOCAB = 100000
EMBED_DIM = 128
MAX_LEN = 200
BATCH = 1024

NUM_WORKERS = 32
ROWS_PER_WORKER = BATCH * MAX_LEN // NUM_WORKERS
CHUNK = 80
CHUNKS_PER_WORKER = ROWS_PER_WORKER // CHUNK
LANES = 16
NBUF = 5
PF = 3


def _emb_kernel(idx_hbm, glove_hbm, pos2_hbm, out_hbm,
                idx_v, pos_v, b0, b1, b2, b3, b4,
                g0, g1, g2, g3, g4, w0, w1, w2, w3, w4):
    bufs = (b0, b1, b2, b3, b4)
    gsems = (g0, g1, g2, g3, g4)
    wsems = (w0, w1, w2, w3, w4)

    wid = lax.axis_index("s") * 2 + lax.axis_index("c")
    pltpu.sync_copy(idx_hbm.at[pl.ds(wid * CHUNKS_PER_WORKER,
                                     CHUNKS_PER_WORKER)], idx_v)
    out_base = wid * ROWS_PER_WORKER

    def issue_gather(c, i):
        pltpu.async_copy(glove_hbm.at[idx_v.at[c]], bufs[i], gsems[i])

    def wait_gather(c, i):
        pltpu.make_async_copy(glove_hbm.at[idx_v.at[c]],
                              bufs[i], gsems[i]).wait()

    def wait_write(i):
        pltpu.make_async_copy(
            bufs[i], out_hbm.at[pl.ds(out_base, CHUNK)], wsems[i]).wait()

    def stage(c, i, pf, wwait):
        j = (i + PF) % NBUF
        if wwait:
            wait_write(j)
        if pf:
            issue_gather(c + PF, j)
        wait_gather(c, i)
        off = (i * CHUNK) % MAX_LEN

        @plsc.parallel_loop(0, CHUNK, unroll=4)
        def _row(r):
            for cc in range(EMBED_DIM // LANES):
                slc = pl.ds(cc * LANES, LANES)
                plsc.addupdate(bufs[i].at[r, slc], pos_v[off + r, slc])

        pltpu.async_copy(
            bufs[i], out_hbm.at[pl.ds(out_base + c * CHUNK, CHUNK)], wsems[i])

    for c in range(PF):
        issue_gather(c, c)
    pltpu.sync_copy(pos2_hbm, pos_v.at[pl.ds(0, MAX_LEN)])
    pltpu.sync_copy(pos2_hbm, pos_v.at[pl.ds(MAX_LEN, MAX_LEN)])
    stage(0, 0, pf=True, wwait=False)
    stage(1, 1, pf=True, wwait=False)
    stage(2, 2, pf=True, wwait=True)

    @pl.loop(0, (CHUNKS_PER_WORKER - 10) // NBUF)
    def _group(q):
        c0 = NBUF * q + 3
        stage(c0, 3, pf=True, wwait=True)
        stage(c0 + 1, 4, pf=True, wwait=True)
        stage(c0 + 2, 0, pf=True, wwait=True)
        stage(c0 + 3, 1, pf=True, wwait=True)
        stage(c0 + 4, 2, pf=True, wwait=True)

    N = CHUNKS_PER_WORKER
    stage(N - 7, 3, pf=True, wwait=True)
    stage(N - 6, 4, pf=True, wwait=True)
    stage(N - 5, 0, pf=True, wwait=True)
    stage(N - 4, 1, pf=True, wwait=True)
    stage(N - 3, 2, pf=False, wwait=False)
    stage(N - 2, 3, pf=False, wwait=False)
    stage(N - 1, 4, pf=False, wwait=False)

    for i in range(NBUF):
        wait_write(i)


@jax.jit
def _embed(idx2d, glove, pos2):
    mesh = plsc.VectorSubcoreMesh(core_axis_name="c", subcore_axis_name="s")
    run = functools.partial(
        pl.kernel,
        out_type=jax.ShapeDtypeStruct((BATCH * MAX_LEN, EMBED_DIM), jnp.float32),
        mesh=mesh,
        scratch_types=(
            [pltpu.VMEM((CHUNKS_PER_WORKER, CHUNK), jnp.int32),
             pltpu.VMEM((2 * MAX_LEN, EMBED_DIM), jnp.float32)]
            + [pltpu.VMEM((CHUNK, EMBED_DIM), jnp.float32)] * NBUF
            + [pltpu.SemaphoreType.DMA] * (2 * NBUF)
        ),
    )(_emb_kernel)
    return run(idx2d, glove, pos2)


def kernel(x, glove, pos_table):
    idx2d = x.astype(jnp.int32).reshape(-1, CHUNK)
    out = _embed(idx2d, glove, pos_table)
    return out.reshape(BATCH, MAX_LEN, EMBED_DIM)


# speedup vs baseline: 1.0346x; 1.0346x over previous
import functools

import jax
import jax.numpy as jnp
from jax import lax
from jax.experimental import pallas as pl
from jax.experimental.pallas import tpu as pltpu
from jax.experimental.pallas import tpu_sc as plsc

VOCAB = 100000
EMBED_DIM = 128
MAX_LEN = 200
BATCH = 1024

NUM_WORKERS = 32
ROWS_PER_WORKER = BATCH * MAX_LEN // NUM_WORKERS
CHUNK = 40
CHUNKS_PER_WORKER = ROWS_PER_WORKER // CHUNK
LANES = 16
NBUF = 10
PF = 7


def _emb_kernel(idx_hbm, glove_hbm, pos_hbm, out_hbm,
                idx_v, pos_v, *bufs_and_sems):
    bufs = bufs_and_sems[:NBUF]
    gsems = bufs_and_sems[NBUF:2 * NBUF]
    wsems = bufs_and_sems[2 * NBUF:]

    wid = lax.axis_index("s") * 2 + lax.axis_index("c")
    pltpu.sync_copy(idx_hbm.at[pl.ds(wid * CHUNKS_PER_WORKER,
                                     CHUNKS_PER_WORKER)], idx_v)
    out_base = wid * ROWS_PER_WORKER

    def issue_gather(c, i):
        pltpu.async_copy(glove_hbm.at[idx_v.at[c]], bufs[i], gsems[i])

    def wait_gather(c, i):
        pltpu.make_async_copy(glove_hbm.at[idx_v.at[c]],
                              bufs[i], gsems[i]).wait()

    def wait_write(i):
        pltpu.make_async_copy(
            bufs[i], out_hbm.at[pl.ds(out_base, CHUNK)], wsems[i]).wait()

    def stage(c, i, pf, wwait):
        j = (i + PF) % NBUF
        if wwait:
            wait_write(j)
        if pf:
            issue_gather(c + PF, j)
        wait_gather(c, i)
        off = (i * CHUNK) % MAX_LEN

        @plsc.parallel_loop(0, CHUNK, unroll=2)
        def _row(r):
            for cc in range(EMBED_DIM // LANES):
                slc = pl.ds(cc * LANES, LANES)
                plsc.addupdate(bufs[i].at[r, slc], pos_v[off + r, slc])

        pltpu.async_copy(
            bufs[i], out_hbm.at[pl.ds(out_base + c * CHUNK, CHUNK)], wsems[i])

    for c in range(PF):
        issue_gather(c, c)
    pltpu.sync_copy(pos_hbm, pos_v)

    for c in range(NBUF):
        stage(c, c, pf=True, wwait=(c >= NBUF - PF))

    N = CHUNKS_PER_WORKER
    n_groups = (N - NBUF - PF) // NBUF

    @pl.loop(0, n_groups)
    def _group(q):
        c0 = NBUF * (q + 1)
        for k in range(NBUF):
            stage(c0 + k, k, pf=True, wwait=True)

    tail0 = NBUF * (n_groups + 1)
    for c in range(tail0, N):
        stage(c, c % NBUF, pf=(c + PF < N), wwait=(c + PF < N))

    for i in range(NBUF):
        wait_write(i)


@jax.jit
def _embed(idx2d, glove, pos_table):
    mesh = plsc.VectorSubcoreMesh(core_axis_name="c", subcore_axis_name="s")
    run = functools.partial(
        pl.kernel,
        out_type=jax.ShapeDtypeStruct((BATCH * MAX_LEN, EMBED_DIM), jnp.float32),
        mesh=mesh,
        scratch_types=(
            [pltpu.VMEM((CHUNKS_PER_WORKER, CHUNK), jnp.int32),
             pltpu.VMEM((MAX_LEN, EMBED_DIM), jnp.float32)]
            + [pltpu.VMEM((CHUNK, EMBED_DIM), jnp.float32)] * NBUF
            + [pltpu.SemaphoreType.DMA] * (2 * NBUF)
        ),
    )(_emb_kernel)
    return run(idx2d, glove, pos_table)


def kernel(x, glove, pos_table):
    idx2d = x.astype(jnp.int32).reshape(-1, CHUNK)
    out = _embed(idx2d, glove, pos_table)
    return out.reshape(BATCH, MAX_LEN, EMBED_DIM)
